# baseline (speedup 1.0000x reference)
import functools

import jax
import jax.numpy as jnp
from jax import lax
from jax.experimental import pallas as pl
from jax.experimental.pallas import tpu as pltpu
from jax.experimental.pallas import tpu_sc as plsc

_N = 100000
_E = 3200000
_C = 16
_LAYERS = 3
_ALPHA = 0.9

_NC = 2
_NS = 16
_IDXW = 128
_JB = 8
_CHUNK = _IDXW * _JB
_CPT = -(-_E // (_NS * _CHUNK))
_EPAD = _CPT * _CHUNK * _NS
_EROWS = _EPAD // _IDXW
_RPT = _EROWS // _NS
_NP = 100096
_HALF = _NP // 2
_DUMP = _HALF
_NACC = _HALF + 8
_APT = _HALF // _NS
_ZROWS = 391

_mesh = plsc.VectorSubcoreMesh(
    core_axis_name="c", subcore_axis_name="s", num_cores=_NC, num_subcores=_NS
)


def _zero_acc(zb, acc, sid):
    def zrow(i, carry):
        zb[i, :] = jnp.zeros((_C,), jnp.float32)
        return carry

    lax.fori_loop(0, _ZROWS, zrow, 0)
    base = sid * _APT
    for k in range(_APT // _ZROWS):
        pltpu.sync_copy(zb, acc.at[pl.ds(base + k * _ZROWS, _ZROWS)])

    @pl.when(sid == 0)
    def _():
        pltpu.sync_copy(zb.at[pl.ds(0, 8)], acc.at[pl.ds(_HALF, 8)])


def _dump_acc(acc, out, c, sid):
    base = sid * _APT
    dst = pl.multiple_of(c * _HALF + base, 8)
    pltpu.sync_copy(acc.at[pl.ds(base, _APT)], out.at[pl.ds(dst, _APT)])


def _issue_cols(col0, col1, colb, c, r0, slot, isem):
    @pl.when(c == 0)
    def _():
        pltpu.async_copy(col0.at[pl.ds(r0, _JB)], colb.at[slot], isem)

    @pl.when(c == 1)
    def _():
        pltpu.async_copy(col1.at[pl.ds(r0, _JB)], colb.at[slot], isem)


@functools.partial(
    pl.kernel,
    out_type=jax.ShapeDtypeStruct((_NP, _C), jnp.float32),
    mesh=_mesh,
    scratch_types=[
        pltpu.VMEM((2, _JB, _IDXW), jnp.int32),
        pltpu.VMEM((_IDXW, _C), jnp.float32),
        pltpu.VMEM((_ZROWS, _C), jnp.float32),
        pltpu.VMEM_SHARED((_NACC, _C), jnp.float32),
        pltpu.SemaphoreType.DMA,
        pltpu.SemaphoreType.DMA,
    ],
    compiler_params=pltpu.CompilerParams(use_tc_tiling_on_sc=False),
)
def _sc_degree(col0, col1, out, colb, ones, zb, acc, isem, ssem):
    c = lax.axis_index("c")
    sid = lax.axis_index("s")

    def orow(i, carry):
        ones[i, :] = jnp.ones((_C,), jnp.float32)
        return carry

    lax.fori_loop(0, _IDXW, orow, 0)
    _zero_acc(zb, acc, sid)
    plsc.subcore_barrier()

    rbase = sid * _RPT
    _issue_cols(col0, col1, colb, c, rbase, 0, isem)

    def chunk(t, carry):
        slot = t & 1
        nslot = 1 - slot

        @pl.when(t > 0)
        def _():
            for j in range(_JB):
                pltpu.make_async_copy(
                    out.at[pl.ds(0, _IDXW)], ones, ssem
                ).wait()

        @pl.when(t + 1 < _CPT)
        def _():
            _issue_cols(col0, col1, colb, c, rbase + (t + 1) * _JB, nslot, isem)

        pltpu.make_async_copy(col0.at[pl.ds(0, _JB)], colb.at[slot], isem).wait()
        for j in range(_JB):
            pltpu.async_copy(ones, acc.at[colb.at[slot, j]], ssem, add=True)
        return carry

    lax.fori_loop(0, _CPT, chunk, 0)
    for j in range(_JB):
        pltpu.make_async_copy(out.at[pl.ds(0, _IDXW)], ones, ssem).wait()
    plsc.subcore_barrier()
    _dump_acc(acc, out, c, sid)


@functools.partial(
    pl.kernel,
    out_type=jax.ShapeDtypeStruct((_NP, _C), jnp.float32),
    mesh=_mesh,
    scratch_types=[
        pltpu.VMEM((2, _JB, _IDXW), jnp.int32),
        pltpu.VMEM((2, _JB, _IDXW), jnp.int32),
        pltpu.VMEM((2, _JB, _IDXW, _C), jnp.float32),
        pltpu.VMEM((_ZROWS, _C), jnp.float32),
        pltpu.VMEM_SHARED((_NACC, _C), jnp.float32),
        pltpu.SemaphoreType.DMA,
        pltpu.SemaphoreType.DMA,
        pltpu.SemaphoreType.DMA,
    ],
    compiler_params=pltpu.CompilerParams(use_tc_tiling_on_sc=False),
)
def _sc_propagate(
    rowp, col0, col1, s, out, rowb, colb, msg, zb, acc, isem, gsem, ssem
):
    c = lax.axis_index("c")
    sid = lax.axis_index("s")

    _zero_acc(zb, acc, sid)
    plsc.subcore_barrier()

    rbase = sid * _RPT

    def issue_idx(t, slot):
        r0 = rbase + t * _JB
        pltpu.async_copy(rowp.at[pl.ds(r0, _JB)], rowb.at[slot], isem)
        _issue_cols(col0, col1, colb, c, r0, slot, isem)

    issue_idx(0, 0)

    def chunk(t, carry):
        slot = t & 1
        nslot = 1 - slot

        @pl.when(t > 0)
        def _():
            for j in range(_JB):
                pltpu.make_async_copy(
                    s.at[pl.ds(0, _IDXW)], msg.at[nslot, j], ssem
                ).wait()

        @pl.when(t + 1 < _CPT)
        def _():
            issue_idx(t + 1, nslot)

        pltpu.make_async_copy(rowp.at[pl.ds(0, _JB)], rowb.at[slot], isem).wait()
        pltpu.make_async_copy(rowp.at[pl.ds(0, _JB)], colb.at[slot], isem).wait()
        for j in range(_JB):
            pltpu.async_copy(s.at[rowb.at[slot, j]], msg.at[slot, j], gsem)
        for j in range(_JB):
            pltpu.make_async_copy(
                s.at[pl.ds(0, _IDXW)], msg.at[slot, j], gsem
            ).wait()
        for j in range(_JB):
            pltpu.async_copy(msg.at[slot, j], acc.at[colb.at[slot, j]], ssem, add=True)
        return carry

    lax.fori_loop(0, _CPT, chunk, 0)
    last = (_CPT - 1) & 1
    for j in range(_JB):
        pltpu.make_async_copy(s.at[pl.ds(0, _IDXW)], msg.at[last, j], ssem).wait()
    plsc.subcore_barrier()
    _dump_acc(acc, out, c, sid)


_IB = 1568


def _idx_body(colp, c0, c1):
    col = colp[...]
    c0[...] = jnp.where(col < _HALF, col, _DUMP)
    c1[...] = jnp.where(col >= _HALF, col - _HALF, _DUMP)


_tc_localize = pl.pallas_call(
    _idx_body,
    grid=(_EROWS // _IB,),
    in_specs=[pl.BlockSpec((_IB, _IDXW), lambda i: (i, 0))],
    out_specs=[
        pl.BlockSpec((_IB, _IDXW), lambda i: (i, 0)),
        pl.BlockSpec((_IB, _IDXW), lambda i: (i, 0)),
    ],
    out_shape=[jax.ShapeDtypeStruct((_EROWS, _IDXW), jnp.int32)] * 2,
)

_PB = 3128


def _prep_body(y, m, d, s0, res, dvb):
    deg = d[...]
    dinv = jnp.where(deg > 0, lax.rsqrt(jnp.maximum(deg, 1.0)), 0.0)
    o = jnp.where(m[...] > 0, y[...], 0.0)
    res[...] = (1.0 - _ALPHA) * o
    s0[...] = dinv * o
    dvb[...] = dinv


_tc_prep = pl.pallas_call(
    _prep_body,
    grid=(_NP // _PB,),
    in_specs=[
        pl.BlockSpec((_PB, _C), lambda i: (i, 0)),
        pl.BlockSpec((_PB, 1), lambda i: (i, 0)),
        pl.BlockSpec((_PB, _C), lambda i: (i, 0)),
    ],
    out_specs=[
        pl.BlockSpec((_PB, _C), lambda i: (i, 0)),
        pl.BlockSpec((_PB, _C), lambda i: (i, 0)),
        pl.BlockSpec((_PB, _C), lambda i: (i, 0)),
    ],
    out_shape=[jax.ShapeDtypeStruct((_NP, _C), jnp.float32)] * 3,
)


def _upd_body(final, q, res, dvb, o):
    x = _ALPHA * dvb[...] * q[...] + res[...]
    x = jnp.clip(x, 0.0, 1.0)
    o[...] = x if final else dvb[...] * x


def _make_update(final):
    return pl.pallas_call(
        functools.partial(_upd_body, final),
        grid=(_NP // _PB,),
        in_specs=[pl.BlockSpec((_PB, _C), lambda i: (i, 0))] * 3,
        out_specs=pl.BlockSpec((_PB, _C), lambda i: (i, 0)),
        out_shape=jax.ShapeDtypeStruct((_NP, _C), jnp.float32),
    )


_tc_update = _make_update(False)
_tc_update_final = _make_update(True)


def kernel(y, adj_t, train_mask):
    row = adj_t[0]
    col = adj_t[1]
    pad = _EPAD - _E
    rowp = jnp.concatenate([row, jnp.zeros((pad,), jnp.int32)])
    colp = jnp.concatenate([col, jnp.full((pad,), _N, jnp.int32)])
    rowp = rowp.reshape(_EROWS, _IDXW)
    colp = colp.reshape(_EROWS, _IDXW)
    col0, col1 = _tc_localize(colp)

    deg = _sc_degree(col0, col1)
    yp = jnp.concatenate([y, jnp.zeros((_NP - _N, _C), jnp.float32)])
    m = jnp.concatenate(
        [train_mask.astype(jnp.int32), jnp.zeros((_NP - _N,), jnp.int32)]
    ).reshape(_NP, 1)
    s, res, dvb = _tc_prep(yp, m, deg)

    out = None
    for layer in range(_LAYERS):
        q = _sc_propagate(rowp, col0, col1, s)
        if layer < _LAYERS - 1:
            s = _tc_update(q, res, dvb)
        else:
            out = _tc_update_final(q, res, dvb)
    return out[:_N]


# speedup vs baseline: 1.2660x; 1.2660x over previous
import functools

import jax
import jax.numpy as jnp
from jax import lax
from jax.experimental import pallas as pl
from jax.experimental.pallas import tpu as pltpu
from jax.experimental.pallas import tpu_sc as plsc

_N = 100000
_E = 3200000
_C = 16
_LAYERS = 3
_ALPHA = 0.9

_NC = 2
_NS = 16
_IDXW = 128
_JB = 8
_CHUNK = _IDXW * _JB
_CPT = -(-_E // (_NS * _CHUNK))
_EPAD = _CPT * _CHUNK * _NS
_EROWS = _EPAD // _IDXW
_RPT = _EROWS // _NS
_NP = 100096
_HALF = _NP // 2
_DUMP = _HALF
_NACC = _HALF + 8
_APT = _HALF // _NS
_ZROWS = 391

_mesh = plsc.VectorSubcoreMesh(
    core_axis_name="c", subcore_axis_name="s", num_cores=_NC, num_subcores=_NS
)


def _zero_acc(zb, acc, sid):
    def zrow(i, carry):
        zb[i, :] = jnp.zeros((_C,), jnp.float32)
        return carry

    lax.fori_loop(0, _ZROWS, zrow, 0)
    base = sid * _APT
    for k in range(_APT // _ZROWS):
        pltpu.sync_copy(zb, acc.at[pl.ds(base + k * _ZROWS, _ZROWS)])

    @pl.when(sid == 0)
    def _():
        pltpu.sync_copy(zb.at[pl.ds(0, 8)], acc.at[pl.ds(_HALF, 8)])


def _dump_acc(acc, out, c, sid):
    base = sid * _APT
    dst = pl.multiple_of(c * _HALF + base, 8)
    pltpu.sync_copy(acc.at[pl.ds(base, _APT)], out.at[pl.ds(dst, _APT)])


def _issue_cols(col0, col1, colb, c, r0, slot, isem):
    @pl.when(c == 0)
    def _():
        pltpu.async_copy(col0.at[pl.ds(r0, _JB)], colb.at[slot], isem)

    @pl.when(c == 1)
    def _():
        pltpu.async_copy(col1.at[pl.ds(r0, _JB)], colb.at[slot], isem)


_NW = _NC * _NS
_RPW = _EROWS // _NW
_CPW = _RPW // _JB


@functools.partial(
    pl.kernel,
    out_type=jax.ShapeDtypeStruct((_NW, _NP), jnp.float32),
    mesh=_mesh,
    scratch_types=[
        pltpu.VMEM((2, _JB, _IDXW), jnp.int32),
        pltpu.VMEM((_NP,), jnp.float32),
        pltpu.SemaphoreType.DMA,
    ],
    compiler_params=pltpu.CompilerParams(
        use_tc_tiling_on_sc=False, needs_layout_passes=False
    ),
)
def _sc_hist(colp, out, colb, hist, isem):
    c = lax.axis_index("c")
    sid = lax.axis_index("s")
    w = c * _NS + sid
    ones16 = jnp.ones((16,), jnp.float32)

    def zrow(i, carry):
        hist[pl.ds(i * 16, 16)] = jnp.zeros((16,), jnp.float32)
        return carry

    lax.fori_loop(0, _NP // 16, zrow, 0)

    rbase = w * _RPW
    pltpu.async_copy(colp.at[pl.ds(rbase, _JB)], colb.at[0], isem)

    def chunk(t, carry):
        slot = t & 1
        nslot = 1 - slot

        @pl.when(t + 1 < _CPW)
        def _():
            pltpu.async_copy(
                colp.at[pl.ds(rbase + (t + 1) * _JB, _JB)], colb.at[nslot], isem
            )

        pltpu.make_async_copy(colp.at[pl.ds(0, _JB)], colb.at[slot], isem).wait()
        for j in range(_JB):
            for v in range(_IDXW // 16):
                idx = colb[slot, j, pl.ds(v * 16, 16)]
                plsc.addupdate_scatter(hist, [idx], ones16)
        return carry

    lax.fori_loop(0, _CPW, chunk, 0)
    pltpu.sync_copy(hist, out.at[w])


@functools.partial(
    pl.kernel,
    out_type=jax.ShapeDtypeStruct((_NP, _C), jnp.float32),
    mesh=_mesh,
    scratch_types=[
        pltpu.VMEM((2, _JB, _IDXW), jnp.int32),
        pltpu.VMEM((2, _JB, _IDXW), jnp.int32),
        pltpu.VMEM((2, _JB, _IDXW, _C), jnp.float32),
        pltpu.VMEM((_ZROWS, _C), jnp.float32),
        pltpu.VMEM_SHARED((_NACC, _C), jnp.float32),
        pltpu.SemaphoreType.DMA,
        pltpu.SemaphoreType.DMA,
        pltpu.SemaphoreType.DMA,
    ],
    compiler_params=pltpu.CompilerParams(use_tc_tiling_on_sc=False),
)
def _sc_propagate(
    rowp, col0, col1, s, out, rowb, colb, msg, zb, acc, isem, gsem, ssem
):
    c = lax.axis_index("c")
    sid = lax.axis_index("s")

    _zero_acc(zb, acc, sid)
    plsc.subcore_barrier()

    rbase = sid * _RPT

    def issue_idx(t, slot):
        r0 = rbase + t * _JB
        pltpu.async_copy(rowp.at[pl.ds(r0, _JB)], rowb.at[slot], isem)
        _issue_cols(col0, col1, colb, c, r0, slot, isem)

    issue_idx(0, 0)

    def chunk(t, carry):
        slot = t & 1
        nslot = 1 - slot

        @pl.when(t > 0)
        def _():
            for j in range(_JB):
                pltpu.make_async_copy(
                    s.at[pl.ds(0, _IDXW)], msg.at[nslot, j], ssem
                ).wait()

        @pl.when(t + 1 < _CPT)
        def _():
            issue_idx(t + 1, nslot)

        pltpu.make_async_copy(rowp.at[pl.ds(0, _JB)], rowb.at[slot], isem).wait()
        pltpu.make_async_copy(rowp.at[pl.ds(0, _JB)], colb.at[slot], isem).wait()
        for j in range(_JB):
            pltpu.async_copy(s.at[rowb.at[slot, j]], msg.at[slot, j], gsem)
        for j in range(_JB):
            pltpu.make_async_copy(
                s.at[pl.ds(0, _IDXW)], msg.at[slot, j], gsem
            ).wait()
        for j in range(_JB):
            pltpu.async_copy(msg.at[slot, j], acc.at[colb.at[slot, j]], ssem, add=True)
        return carry

    lax.fori_loop(0, _CPT, chunk, 0)
    last = (_CPT - 1) & 1
    for j in range(_JB):
        pltpu.make_async_copy(s.at[pl.ds(0, _IDXW)], msg.at[last, j], ssem).wait()
    plsc.subcore_barrier()
    _dump_acc(acc, out, c, sid)


_IB = 1568


def _idx_body(colp, c0, c1):
    col = colp[...]
    c0[...] = jnp.where(col < _HALF, col, _DUMP)
    c1[...] = jnp.where(col >= _HALF, col - _HALF, _DUMP)


_tc_localize = pl.pallas_call(
    _idx_body,
    grid=(_EROWS // _IB,),
    in_specs=[pl.BlockSpec((_IB, _IDXW), lambda i: (i, 0))],
    out_specs=[
        pl.BlockSpec((_IB, _IDXW), lambda i: (i, 0)),
        pl.BlockSpec((_IB, _IDXW), lambda i: (i, 0)),
    ],
    out_shape=[jax.ShapeDtypeStruct((_EROWS, _IDXW), jnp.int32)] * 2,
)

_RB = 5888


def _red_body(p, o):
    o[...] = jnp.sum(p[...], axis=0, keepdims=True)


_tc_reduce = pl.pallas_call(
    _red_body,
    grid=(_NP // _RB,),
    in_specs=[pl.BlockSpec((_NW, _RB), lambda i: (0, i))],
    out_specs=pl.BlockSpec((1, _RB), lambda i: (0, i)),
    out_shape=jax.ShapeDtypeStruct((1, _NP), jnp.float32),
)

_PB = 3128


def _prep_body(y, m, d, s0, res, dvb):
    deg = d[...]
    dinv = jnp.where(deg > 0, lax.rsqrt(jnp.maximum(deg, 1.0)), 0.0)
    o = jnp.where(m[...] > 0, y[...], 0.0)
    res[...] = (1.0 - _ALPHA) * o
    s0[...] = dinv * o
    dvb[...] = jnp.broadcast_to(dinv, (_PB, _C))


_tc_prep = pl.pallas_call(
    _prep_body,
    grid=(_NP // _PB,),
    in_specs=[
        pl.BlockSpec((_PB, _C), lambda i: (i, 0)),
        pl.BlockSpec((_PB, 1), lambda i: (i, 0)),
        pl.BlockSpec((_PB, 1), lambda i: (i, 0)),
    ],
    out_specs=[
        pl.BlockSpec((_PB, _C), lambda i: (i, 0)),
        pl.BlockSpec((_PB, _C), lambda i: (i, 0)),
        pl.BlockSpec((_PB, _C), lambda i: (i, 0)),
    ],
    out_shape=[jax.ShapeDtypeStruct((_NP, _C), jnp.float32)] * 3,
)


def _upd_body(final, q, res, dvb, o):
    x = _ALPHA * dvb[...] * q[...] + res[...]
    x = jnp.clip(x, 0.0, 1.0)
    o[...] = x if final else dvb[...] * x


def _make_update(final):
    return pl.pallas_call(
        functools.partial(_upd_body, final),
        grid=(_NP // _PB,),
        in_specs=[pl.BlockSpec((_PB, _C), lambda i: (i, 0))] * 3,
        out_specs=pl.BlockSpec((_PB, _C), lambda i: (i, 0)),
        out_shape=jax.ShapeDtypeStruct((_NP, _C), jnp.float32),
    )


_tc_update = _make_update(False)
_tc_update_final = _make_update(True)


def kernel(y, adj_t, train_mask):
    row = adj_t[0]
    col = adj_t[1]
    pad = _EPAD - _E
    rowp = jnp.concatenate([row, jnp.zeros((pad,), jnp.int32)])
    colp = jnp.concatenate([col, jnp.full((pad,), _N, jnp.int32)])
    rowp = rowp.reshape(_EROWS, _IDXW)
    colp = colp.reshape(_EROWS, _IDXW)
    col0, col1 = _tc_localize(colp)

    deg = _tc_reduce(_sc_hist(colp)).reshape(_NP, 1)
    yp = jnp.concatenate([y, jnp.zeros((_NP - _N, _C), jnp.float32)])
    m = jnp.concatenate(
        [train_mask.astype(jnp.int32), jnp.zeros((_NP - _N,), jnp.int32)]
    ).reshape(_NP, 1)
    s, res, dvb = _tc_prep(yp, m, deg)

    out = None
    for layer in range(_LAYERS):
        q = _sc_propagate(rowp, col0, col1, s)
        if layer < _LAYERS - 1:
            s = _tc_update(q, res, dvb)
        else:
            out = _tc_update_final(q, res, dvb)
    return out[:_N]
